# baseline (speedup 1.0000x reference)
import jax
import jax.numpy as jnp
from jax.experimental import pallas as pl
from jax.experimental.pallas import tpu as pltpu

N = 10000
NFEAT = 128
NHID = 128
NCLASS = 16
SHID = N // 3

BM = 200
NBM = N // BM
BH2 = 128
NW2 = pl.cdiv(SHID, BH2)
P1_END = NBM + NW2
GRID = NBM + NW2 + NBM
HPAD = NW2 * BH2


def _mega_kernel(adj_ref, x_ref, wg_ref, bg_ref, wcat_ref, w1_ref, b1_ref,
                 w2_ref, b2_ref, eps_ref, b11_ref, b12_ref, out_ref,
                 m_ref, hacc_ref, h_ref, sc_ref):
    i = pl.program_id(0)

    @pl.when(i < NBM)
    def _():
        t = jax.lax.dot_general(
            adj_ref[...], x_ref[...], (((1,), (0,)), ((), ())),
            preferred_element_type=jnp.float32)
        hgc = jax.lax.dot_general(
            t, wg_ref[...], (((1,), (0,)), ((), ())),
            preferred_element_type=jnp.float32)
        hgc = jax.nn.relu(hgc + bg_ref[...])
        m_ref[pl.ds(i * BM, BM), :] = jax.lax.dot_general(
            hgc, wcat_ref[...], (((1,), (0,)), ((), ())),
            preferred_element_type=jnp.float32)

        s_blk = jnp.sum(x_ref[pl.ds(i * BM, BM), :], axis=1, keepdims=True)
        part = jnp.sum(w1_ref[...] * s_blk, axis=0, keepdims=True)
        part = jnp.concatenate(
            [part, jnp.zeros((1, HPAD - SHID), jnp.float32)], axis=1)

        @pl.when(i == 0)
        def _():
            hacc_ref[...] = part

        @pl.when(i > 0)
        def _():
            hacc_ref[...] = hacc_ref[...] + part

        @pl.when(i == NBM - 1)
        def _():
            h_ref[...] = jax.nn.relu(
                hacc_ref[...] + b1_ref[...]).reshape(HPAD, 1)

    @pl.when(jnp.logical_and(i >= NBM, i < P1_END))
    def _():
        j = i - NBM
        hc = h_ref[pl.ds(j * BH2, BH2), :]
        row = jax.lax.broadcasted_iota(jnp.int32, (BH2, 1), 0) + j * BH2
        prod = jnp.where(row < SHID, w2_ref[...] * hc, 0.0)
        part = jnp.sum(prod, axis=0, keepdims=True)

        @pl.when(j == 0)
        def _():
            sc_ref[...] = part

        @pl.when(j > 0)
        def _():
            sc_ref[...] = sc_ref[...] + part

        @pl.when(j == NW2 - 1)
        def _():
            sc_ref[...] = jax.nn.sigmoid(sc_ref[...] + b2_ref[...])

    @pl.when(i >= P1_END)
    def _():
        adj_s = adj_ref[...] * sc_ref[...]
        acc = jax.lax.dot_general(
            adj_s, m_ref[...], (((1,), (0,)), ((), ())),
            preferred_element_type=jnp.float32)
        mean = acc[:, :NCLASS] + b11_ref[...]
        logstd = acc[:, NCLASS:] + b12_ref[...]
        z = eps_ref[...] * jnp.exp(logstd) + mean
        zmax = jnp.max(z, axis=1, keepdims=True)
        ze = z - zmax
        out_ref[...] = ze - jnp.log(
            jnp.sum(jnp.exp(ze), axis=1, keepdims=True))


def kernel(x, adj, W_gc1, b_gc1, W_fc11, b_fc11, W_fc12, b_fc12,
           W_se1, b_se1, W_se2, b_se2, eps):
    f32 = jnp.float32
    wcat = jnp.concatenate([W_fc11, W_fc12], axis=1)

    out = pl.pallas_call(
        _mega_kernel,
        grid=(GRID,),
        in_specs=[
            pl.BlockSpec((BM, N),
                         lambda i: (jnp.where(i < NBM, i,
                                              jnp.maximum(i - P1_END, 0)), 0)),
            pl.BlockSpec((N, NFEAT), lambda i: (0, 0)),
            pl.BlockSpec((NFEAT, NHID), lambda i: (0, 0)),
            pl.BlockSpec((1, NHID), lambda i: (0, 0)),
            pl.BlockSpec((NHID, 2 * NCLASS), lambda i: (0, 0)),
            pl.BlockSpec((BM, SHID),
                         lambda i: (jnp.minimum(i, NBM - 1), 0)),
            pl.BlockSpec((1, HPAD), lambda i: (0, 0)),
            pl.BlockSpec((BH2, N),
                         lambda i: (jnp.clip(i - NBM, 0, NW2 - 1), 0)),
            pl.BlockSpec((1, N), lambda i: (0, 0)),
            pl.BlockSpec((BM, NCLASS),
                         lambda i: (jnp.maximum(i - P1_END, 0), 0)),
            pl.BlockSpec((1, NCLASS), lambda i: (0, 0)),
            pl.BlockSpec((1, NCLASS), lambda i: (0, 0)),
        ],
        out_specs=pl.BlockSpec((BM, NCLASS),
                               lambda i: (jnp.maximum(i - P1_END, 0), 0)),
        out_shape=jax.ShapeDtypeStruct((N, NCLASS), f32),
        scratch_shapes=[
            pltpu.VMEM((N, 2 * NCLASS), f32),
            pltpu.VMEM((1, HPAD), f32),
            pltpu.VMEM((HPAD, 1), f32),
            pltpu.VMEM((1, N), f32),
        ],
        compiler_params=pltpu.CompilerParams(
            dimension_semantics=("arbitrary",)),
    )(adj, x, W_gc1, b_gc1.reshape(1, NHID), wcat, W_se1,
      jnp.pad(b_se1, (0, HPAD - SHID)).reshape(1, HPAD),
      W_se2, b_se2.reshape(1, N), eps,
      b_fc11.reshape(1, NCLASS), b_fc12.reshape(1, NCLASS))

    return out


# speedup vs baseline: 1.0156x; 1.0156x over previous
import jax
import jax.numpy as jnp
from jax.experimental import pallas as pl
from jax.experimental.pallas import tpu as pltpu

N = 10000
NFEAT = 128
NHID = 128
NCLASS = 16
SHID = N // 3

BX = 1000
NB1 = N // BX
BH = 384
NB2 = pl.cdiv(SHID, BH)
BM = 400
NBM = N // BM


def _a1_kernel(x_ref, wg_ref, w1_ref, b1_ref, xw_ref, h_ref, hacc_ref):
    i = pl.program_id(0)
    x = x_ref[...]
    xw_ref[...] = jax.lax.dot_general(
        x, wg_ref[...], (((1,), (0,)), ((), ())),
        preferred_element_type=jnp.float32)
    s = jnp.sum(x, axis=1, keepdims=True)
    part = jnp.sum(w1_ref[...] * s, axis=0, keepdims=True)

    @pl.when(i == 0)
    def _():
        hacc_ref[...] = part

    @pl.when(i > 0)
    def _():
        hacc_ref[...] = hacc_ref[...] + part

    @pl.when(i == NB1 - 1)
    def _():
        h_ref[...] = jax.nn.relu(hacc_ref[...] + b1_ref[...])


def _a2_kernel(h_ref, w2_ref, b2_ref, sc_ref):
    i = pl.program_id(0)
    hc = h_ref[...].reshape(BH, 1)
    row = jax.lax.broadcasted_iota(jnp.int32, (BH, 1), 0) + i * BH
    prod = jnp.where(row < SHID, w2_ref[...] * hc, 0.0)
    part = jnp.sum(prod, axis=0, keepdims=True)

    @pl.when(i == 0)
    def _():
        sc_ref[...] = part

    @pl.when(i > 0)
    def _():
        sc_ref[...] = sc_ref[...] + part

    @pl.when(i == NB2 - 1)
    def _():
        sc_ref[...] = jax.nn.sigmoid(sc_ref[...] + b2_ref[...])


def _bc_kernel(adj_ref, xw_ref, bg_ref, wcat_ref, score_ref, eps_ref,
               b11_ref, b12_ref, out_ref, m_ref):
    i = pl.program_id(0)

    @pl.when(i < NBM)
    def _():
        h = jax.lax.dot_general(
            adj_ref[...], xw_ref[...], (((1,), (0,)), ((), ())),
            preferred_element_type=jnp.float32)
        h = jax.nn.relu(h + bg_ref[...])
        hw = jax.lax.dot_general(
            h, wcat_ref[...], (((1,), (0,)), ((), ())),
            preferred_element_type=jnp.float32)
        m_ref[pl.ds(i * BM, BM), :] = hw

    @pl.when(i >= NBM)
    def _():
        adj_s = adj_ref[...] * score_ref[...]
        acc = jax.lax.dot_general(
            adj_s, m_ref[...], (((1,), (0,)), ((), ())),
            preferred_element_type=jnp.float32)
        mean = acc[:, :NCLASS] + b11_ref[...]
        logstd = acc[:, NCLASS:] + b12_ref[...]
        z = eps_ref[...] * jnp.exp(logstd) + mean
        zmax = jnp.max(z, axis=1, keepdims=True)
        ze = z - zmax
        out_ref[...] = ze - jnp.log(
            jnp.sum(jnp.exp(ze), axis=1, keepdims=True))


def kernel(x, adj, W_gc1, b_gc1, W_fc11, b_fc11, W_fc12, b_fc12,
           W_se1, b_se1, W_se2, b_se2, eps):
    f32 = jnp.float32

    xw, h = pl.pallas_call(
        _a1_kernel,
        grid=(NB1,),
        in_specs=[
            pl.BlockSpec((BX, NFEAT), lambda i: (i, 0)),
            pl.BlockSpec((NFEAT, NHID), lambda i: (0, 0)),
            pl.BlockSpec((BX, SHID), lambda i: (i, 0)),
            pl.BlockSpec((1, SHID), lambda i: (0, 0)),
        ],
        out_specs=[
            pl.BlockSpec((BX, NHID), lambda i: (i, 0)),
            pl.BlockSpec((1, SHID), lambda i: (0, 0)),
        ],
        out_shape=[
            jax.ShapeDtypeStruct((N, NHID), f32),
            jax.ShapeDtypeStruct((1, SHID), f32),
        ],
        scratch_shapes=[pltpu.VMEM((1, SHID), f32)],
        compiler_params=pltpu.CompilerParams(
            dimension_semantics=("arbitrary",)),
    )(x, W_gc1, W_se1, b_se1.reshape(1, SHID))

    sc_row = pl.pallas_call(
        _a2_kernel,
        grid=(NB2,),
        in_specs=[
            pl.BlockSpec((1, BH), lambda i: (0, i)),
            pl.BlockSpec((BH, N), lambda i: (i, 0)),
            pl.BlockSpec((1, N), lambda i: (0, 0)),
        ],
        out_specs=pl.BlockSpec((1, N), lambda i: (0, 0)),
        out_shape=jax.ShapeDtypeStruct((1, N), f32),
        compiler_params=pltpu.CompilerParams(
            dimension_semantics=("arbitrary",)),
    )(h, W_se2, b_se2.reshape(1, N))

    wcat = jnp.concatenate([W_fc11, W_fc12], axis=1)

    out = pl.pallas_call(
        _bc_kernel,
        grid=(2 * NBM,),
        in_specs=[
            pl.BlockSpec((BM, N), lambda i: (jax.lax.rem(i, NBM), 0)),
            pl.BlockSpec((N, NHID), lambda i: (0, 0)),
            pl.BlockSpec((1, NHID), lambda i: (0, 0)),
            pl.BlockSpec((NHID, 2 * NCLASS), lambda i: (0, 0)),
            pl.BlockSpec((1, N), lambda i: (0, 0)),
            pl.BlockSpec((BM, NCLASS), lambda i: (jax.lax.rem(i, NBM), 0)),
            pl.BlockSpec((1, NCLASS), lambda i: (0, 0)),
            pl.BlockSpec((1, NCLASS), lambda i: (0, 0)),
        ],
        out_specs=pl.BlockSpec((BM, NCLASS),
                               lambda i: (jax.lax.rem(i, NBM), 0)),
        out_shape=jax.ShapeDtypeStruct((N, NCLASS), f32),
        scratch_shapes=[pltpu.VMEM((N, 2 * NCLASS), f32)],
        compiler_params=pltpu.CompilerParams(
            dimension_semantics=("arbitrary",)),
    )(adj, xw, b_gc1.reshape(1, NHID), wcat, sc_row, eps,
      b_fc11.reshape(1, NCLASS), b_fc12.reshape(1, NCLASS))

    return out
